# baseline (speedup 1.0000x reference)
import functools

import jax
import jax.numpy as jnp
from jax.experimental import pallas as pl
from jax.experimental.pallas import tpu as pltpu
from jax.experimental.pallas import tpu_sc as plsc

_EMBED_DIM = 64
_PAD_DIM = 128
_NUM_CODES = 1024
_B = 16
_S = 576
_N_TOK = _B * _S
_HALF = _N_TOK // 2
_HROWS = _HALF // 128

_NC = 2
_NS = 16
_NW = _NC * _NS
_RPW = 2


def _score_and_pick(z_ref, wn_ref, idx_ref):
    zt = z_ref[...]
    zn = zt / jnp.maximum(
        jnp.sqrt(jnp.sum(zt * zt, axis=1, keepdims=True)), 1e-12)
    scores = jax.lax.dot_general(
        zn, wn_ref[...], (((1,), (1,)), ((), ())),
        preferred_element_type=jnp.float32)
    m = jnp.max(scores, axis=1, keepdims=True)
    ids = jax.lax.broadcasted_iota(
        jnp.int32, (1, _NUM_CODES), 1).astype(jnp.float32)
    idx = jnp.min(jnp.where(scores == m, ids, jnp.float32(4096.0)), axis=1)
    idx_ref[...] = idx.astype(jnp.int32).reshape(_HROWS, 128)


def _idx_body_w(z_ref, w_ref, idx_ref, wdup_ref, wn_ref):
    w = w_ref[...]
    wn_ref[...] = w / jnp.maximum(
        jnp.sqrt(jnp.sum(w * w, axis=1, keepdims=True)), 1e-12)
    wdup_ref[:, :_EMBED_DIM] = w
    wdup_ref[:, _EMBED_DIM:] = w
    _score_and_pick(z_ref, wn_ref, idx_ref)


def _idx_body(z_ref, w_ref, idx_ref, wn_ref):
    w = w_ref[...]
    wn_ref[...] = w / jnp.maximum(
        jnp.sqrt(jnp.sum(w * w, axis=1, keepdims=True)), 1e-12)
    _score_and_pick(z_ref, wn_ref, idx_ref)


def _gather_body(wdup_hbm, idx_hbm, out_hbm, idx_v, rows_v, sems):
    wid = jax.lax.axis_index("s") * _NC + jax.lax.axis_index("c")
    rows = [wid, wid + _NW]

    for j, r in enumerate(rows):
        @pl.when(r < _HROWS)
        def _load(r=r, j=j):
            pltpu.sync_copy(idx_hbm.at[pl.ds(r, 1), :], idx_v.at[pl.ds(j, 1)])
    for j, r in enumerate(rows):
        @pl.when(r < _HROWS)
        def _fire(r=r, j=j):
            pltpu.async_copy(wdup_hbm.at[idx_v.at[j]], rows_v.at[j], sems[j])
    for j, r in enumerate(rows):
        @pl.when(r < _HROWS)
        def _drain(r=r, j=j):
            pltpu.make_async_copy(
                wdup_hbm.at[idx_v.at[j]], rows_v.at[j], sems[j]).wait()
            pltpu.sync_copy(
                rows_v.at[j, :, pl.ds(0, _EMBED_DIM)],
                out_hbm.at[pl.ds(r * 128, 128)])


_sc_gather = pl.kernel(
    _gather_body,
    out_type=jax.ShapeDtypeStruct((_HALF, _EMBED_DIM), jnp.float32),
    mesh=plsc.VectorSubcoreMesh(
        core_axis_name="c", subcore_axis_name="s",
        num_cores=_NC, num_subcores=_NS),
    scratch_types=[
        pltpu.VMEM((_RPW, 128), jnp.int32),
        pltpu.VMEM((_RPW, 128, _PAD_DIM), jnp.float32),
        [pltpu.SemaphoreType.DMA] * _RPW,
    ],
    compiler_params=pltpu.CompilerParams(use_tc_tiling_on_sc=False),
)


def _tc_call(body, n_out, zh, W):
    outs = [jax.ShapeDtypeStruct((_HROWS, 128), jnp.int32)]
    specs = [pl.BlockSpec((_HROWS, 128), lambda: (0, 0))]
    if n_out == 2:
        outs.append(jax.ShapeDtypeStruct((_NUM_CODES, _PAD_DIM), jnp.float32))
        specs.append(pl.BlockSpec((_NUM_CODES, _PAD_DIM), lambda: (0, 0)))
    res = pl.pallas_call(
        body,
        in_specs=[
            pl.BlockSpec((_HALF, _EMBED_DIM), lambda: (0, 0)),
            pl.BlockSpec((_NUM_CODES, _EMBED_DIM), lambda: (0, 0)),
        ],
        out_specs=specs,
        out_shape=outs,
        scratch_shapes=[pltpu.VMEM((_NUM_CODES, _EMBED_DIM), jnp.float32)],
    )(zh, W)
    return res if n_out == 2 else res[0]


@jax.jit
def kernel(z, W):
    z2 = z.reshape(_N_TOK, _EMBED_DIM)
    idx_a, wdup = _tc_call(_idx_body_w, 2, z2[:_HALF], W)
    idx_b = _tc_call(_idx_body, 1, z2[_HALF:], W)
    out_a = _sc_gather(wdup, idx_a)
    out_b = _sc_gather(wdup, idx_b)
    zq = jnp.concatenate([out_a, out_b], axis=0)
    return zq.reshape(_B, _S, _EMBED_DIM)


# speedup vs baseline: 1.3868x; 1.3868x over previous
import functools

import jax
import jax.numpy as jnp
from jax.experimental import pallas as pl
from jax.experimental.pallas import tpu as pltpu
from jax.experimental.pallas import tpu_sc as plsc

_EMBED_DIM = 64
_NUM_CODES = 1024
_B = 16
_S = 576
_N_TOK = _B * _S
_RPS = 8
_TILE = _RPS * _S
_STEPS = _B // _RPS

_NC = 2
_NS = 16
_NW = _NC * _NS
_BPW = _N_TOK // _NW
_WPB = _NW // _B


def _idx_body(z_ref, w_ref, idx_ref, wn_ref):
    @pl.when(pl.program_id(0) == 0)
    def _init():
        w = w_ref[...]
        wn_ref[...] = w / jnp.maximum(
            jnp.sqrt(jnp.sum(w * w, axis=1, keepdims=True)), 1e-12)

    zt = z_ref[...].reshape(_TILE, _EMBED_DIM)
    zn = zt / jnp.maximum(
        jnp.sqrt(jnp.sum(zt * zt, axis=1, keepdims=True)), 1e-12)
    scores = jax.lax.dot_general(
        zn, wn_ref[...], (((1,), (1,)), ((), ())),
        preferred_element_type=jnp.float32)
    m = jnp.max(scores, axis=1, keepdims=True)
    ids = jax.lax.broadcasted_iota(
        jnp.int32, scores.shape, 1).astype(jnp.float32)
    idx = jnp.min(jnp.where(scores == m, ids, jnp.float32(4096.0)), axis=1)
    i = pl.program_id(0)
    idx_ref[pl.ds(i * _RPS, _RPS), :] = (
        idx.astype(jnp.int32).reshape(_RPS, _S))


def _gather_body(w_hbm, idx_hbm, out_hbm, idx_v, rows_v, sem):
    wid = jax.lax.axis_index("s") * _NC + jax.lax.axis_index("c")
    b = wid // _WPB
    col = (wid % _WPB) * _BPW
    pltpu.sync_copy(idx_hbm.at[b, pl.ds(col, _BPW)], idx_v)
    pltpu.async_copy(w_hbm.at[idx_v], rows_v, sem).wait()
    pltpu.sync_copy(rows_v, out_hbm.at[b, pl.ds(col, _BPW)])


_sc_gather = pl.kernel(
    _gather_body,
    out_type=jax.ShapeDtypeStruct((_B, _S, _EMBED_DIM), jnp.float32),
    mesh=plsc.VectorSubcoreMesh(
        core_axis_name="c", subcore_axis_name="s",
        num_cores=_NC, num_subcores=_NS),
    scratch_types=[
        pltpu.VMEM((_BPW,), jnp.int32),
        pltpu.VMEM((_BPW, _EMBED_DIM), jnp.float32),
        pltpu.SemaphoreType.DMA,
    ],
    compiler_params=pltpu.CompilerParams(use_tc_tiling_on_sc=False),
)


@jax.jit
def kernel(z, W):
    idx = pl.pallas_call(
        _idx_body,
        grid=(_STEPS,),
        in_specs=[
            pl.BlockSpec((_RPS, _S, _EMBED_DIM), lambda i: (i, 0, 0)),
            pl.BlockSpec((_NUM_CODES, _EMBED_DIM), lambda i: (0, 0)),
        ],
        out_specs=pl.BlockSpec((_B, _S), lambda i: (0, 0)),
        out_shape=jax.ShapeDtypeStruct((_B, _S), jnp.int32),
        scratch_shapes=[pltpu.VMEM((_NUM_CODES, _EMBED_DIM), jnp.float32)],
    )(z, W)
    return _sc_gather(W, idx)


# speedup vs baseline: 1.3908x; 1.0029x over previous
import functools

import jax
import jax.numpy as jnp
from jax.experimental import pallas as pl
from jax.experimental.pallas import tpu as pltpu
from jax.experimental.pallas import tpu_sc as plsc

_EMBED_DIM = 64
_NUM_CODES = 1024
_B = 16
_S = 576
_N_TOK = _B * _S
_TILE = _N_TOK // 2
_HROWS = _TILE // 128

_NC = 2
_NS = 16
_NW = _NC * _NS


def _idx_body(z_ref, w_ref, idx0_ref, idx1_ref, wn_ref):
    @pl.when(pl.program_id(0) == 0)
    def _init():
        w = w_ref[...]
        wn_ref[...] = w / jnp.maximum(
            jnp.sqrt(jnp.sum(w * w, axis=1, keepdims=True)), 1e-12)

    zt = z_ref[...]
    zn = zt / jnp.maximum(
        jnp.sqrt(jnp.sum(zt * zt, axis=1, keepdims=True)), 1e-12)
    scores = jax.lax.dot_general(
        zn, wn_ref[...], (((1,), (1,)), ((), ())),
        preferred_element_type=jnp.float32)
    m = jnp.max(scores, axis=1, keepdims=True)
    ids = jax.lax.broadcasted_iota(
        jnp.int32, scores.shape, 1).astype(jnp.float32)
    idx = jnp.min(jnp.where(scores == m, ids, jnp.float32(4096.0)), axis=1)
    packed = idx.astype(jnp.int32).reshape(_HROWS, 128)
    i = pl.program_id(0)

    @pl.when(i == 0)
    def _w0():
        idx0_ref[...] = packed

    @pl.when(i == 1)
    def _w1():
        idx1_ref[...] = packed


def _gather_body(w_hbm, idx0_hbm, idx1_hbm, out_hbm, idx_v, rows_v, sems):
    wid = jax.lax.axis_index("s") * _NC + jax.lax.axis_index("c")

    def loads():
        pltpu.sync_copy(idx0_hbm.at[pl.ds(wid, 1), :], idx_v.at[pl.ds(0, 1)])

        @pl.when(wid < 4)
        def _a():
            pltpu.sync_copy(idx0_hbm.at[pl.ds(wid + 32, 1), :],
                            idx_v.at[pl.ds(1, 1)])

        @pl.when(wid >= 4)
        def _b():
            pltpu.sync_copy(idx1_hbm.at[pl.ds(wid - 4, 1), :],
                            idx_v.at[pl.ds(1, 1)])

        @pl.when(wid < 8)
        def _c():
            pltpu.sync_copy(idx1_hbm.at[pl.ds(wid + 28, 1), :],
                            idx_v.at[pl.ds(2, 1)])

    def fire(j):
        pltpu.async_copy(w_hbm.at[idx_v.at[j]], rows_v.at[j], sems[j])

    def drain(j, row):
        pltpu.make_async_copy(
            w_hbm.at[idx_v.at[j]], rows_v.at[j], sems[j]).wait()
        pltpu.sync_copy(rows_v.at[j], out_hbm.at[pl.ds(row * 128, 128)])

    loads()
    fire(0)
    fire(1)

    @pl.when(wid < 8)
    def _f2():
        fire(2)

    drain(0, wid)
    drain(1, wid + _NW)

    @pl.when(wid < 8)
    def _d2():
        drain(2, wid + 2 * _NW)


_sc_gather = pl.kernel(
    _gather_body,
    out_type=jax.ShapeDtypeStruct((_N_TOK, _EMBED_DIM), jnp.float32),
    mesh=plsc.VectorSubcoreMesh(
        core_axis_name="c", subcore_axis_name="s",
        num_cores=_NC, num_subcores=_NS),
    scratch_types=[
        pltpu.VMEM((3, 128), jnp.int32),
        pltpu.VMEM((3, 128, _EMBED_DIM), jnp.float32),
        [pltpu.SemaphoreType.DMA] * 3,
    ],
    compiler_params=pltpu.CompilerParams(use_tc_tiling_on_sc=False),
)


@jax.jit
def kernel(z, W):
    z2 = z.reshape(_N_TOK, _EMBED_DIM)
    idx0, idx1 = pl.pallas_call(
        _idx_body,
        grid=(2,),
        in_specs=[
            pl.BlockSpec((_TILE, _EMBED_DIM), lambda i: (i, 0)),
            pl.BlockSpec((_NUM_CODES, _EMBED_DIM), lambda i: (0, 0)),
        ],
        out_specs=[
            pl.BlockSpec((_HROWS, 128), lambda i: (0, 0)),
            pl.BlockSpec((_HROWS, 128), lambda i: (0, 0)),
        ],
        out_shape=[
            jax.ShapeDtypeStruct((_HROWS, 128), jnp.int32),
            jax.ShapeDtypeStruct((_HROWS, 128), jnp.int32),
        ],
        scratch_shapes=[pltpu.VMEM((_NUM_CODES, _EMBED_DIM), jnp.float32)],
    )(z2, W)
    return _sc_gather(W, idx0, idx1).reshape(_B, _S, _EMBED_DIM)
